# baseline (speedup 1.0000x reference)
import jax
import jax.numpy as jnp
from jax import lax
from jax.experimental import pallas as pl
from jax.experimental.pallas import tpu as pltpu
from jax.experimental.pallas import tpu_sc as plsc

N = 100000
E = 6400000
NC = 2
NS = 16
NPAD = 100352
RN = NPAD // NS
PADC = NPAD - N

EPT1 = E // NS
C1 = 20000
NCH1 = EPT1 // C1

EPT2 = E // (NC * NS)
C2 = 4000
NCH2 = EPT2 // C2

F1 = 60
F2 = 30
F3 = 10


def _f16_bits(h):
    b = lax.bitcast_convert_type(h, jnp.int32)
    sgn = lax.shift_right_logical(b, 16) & jnp.int32(0x8000)
    mag = b & jnp.int32(0x7FFFFFFF)
    lsb = lax.shift_right_logical(mag, 13) & 1
    rounded = mag + jnp.int32(0xFFF) + lsb
    r16 = lax.shift_right_logical(rounded, 13) - jnp.int32((127 - 15) << 10)
    r16 = jnp.where(r16 <= 0, jnp.int32(0), r16)
    return sgn | r16


def _f16_to_f32(b16):
    sgn = lax.shift_left(b16 & jnp.int32(0x8000), 16)
    rest = b16 & jnp.int32(0x7FFF)
    f32b = jnp.where(rest == 0, jnp.int32(0),
                     lax.shift_left(rest + jnp.int32(112 << 10), 13))
    return lax.bitcast_convert_type(sgn | f32b, jnp.float32)


def _rsqrt_newton(d):
    i = lax.bitcast_convert_type(d, jnp.int32)
    i = jnp.int32(0x5F3759DF) - lax.shift_right_logical(i, 1)
    y = lax.bitcast_convert_type(i, jnp.float32)
    for _ in range(3):
        y = y * (jnp.float32(1.5) - jnp.float32(0.5) * d * y * y)
    return y


def _k1_body(ei_hbm, x_hbm, zeros_hbm, ones_hbm,
             h_out, indeg_out,
             deg_sh, idx_a, idx_b, ones_buf, deg_buf, x_buf, h_buf,
             dma_sem_a, dma_sem_b, sc_sem_a, sc_sem_b):
    c = lax.axis_index("c")
    s = lax.axis_index("s")
    sl = pl.ds(s * RN, RN)
    pltpu.sync_copy(zeros_hbm.at[sl], deg_sh.at[sl])
    pltpu.sync_copy(ones_hbm, ones_buf)
    plsc.subcore_barrier()

    base = s * EPT1
    idx_bufs = [idx_a, idx_b]
    dma_sems = [dma_sem_a, dma_sem_b]
    sc_sems = [sc_sem_a, sc_sem_b]
    dma_descs = [None, None]
    sc_descs = [None, None]
    dma_descs[0] = pltpu.async_copy(
        ei_hbm.at[pl.ds(c * E + base, C1)], idx_bufs[0], dma_sems[0])
    for k in range(NCH1):
        b = k % 2
        if k + 1 < NCH1:
            nb = (k + 1) % 2
            dma_descs[nb] = pltpu.async_copy(
                ei_hbm.at[pl.ds(c * E + base + (k + 1) * C1, C1)],
                idx_bufs[nb], dma_sems[nb])
        dma_descs[b].wait()
        if sc_descs[b] is not None:
            sc_descs[b].wait()
        sc_descs[b] = pltpu.async_copy(
            ones_buf, deg_sh.at[idx_bufs[b]], sc_sems[b], add=True)
    for d in sc_descs:
        if d is not None:
            d.wait()
    plsc.subcore_barrier()

    @pl.when(c == 0)
    def _():
        pltpu.sync_copy(deg_sh.at[sl], deg_buf)
        pltpu.sync_copy(x_hbm.at[sl], x_buf)

        def body(i, carry):
            va = pl.ds(i * 32, 16)
            vb = pl.ds(i * 32 + 16, 16)
            da = jnp.maximum(deg_buf[va], jnp.float32(1.0))
            db = jnp.maximum(deg_buf[vb], jnp.float32(1.0))
            ha = x_buf[va] * _rsqrt_newton(da)
            hb = x_buf[vb] * _rsqrt_newton(db)
            h_buf[pl.ds(i * 16, 16)] = _f16_bits(ha) | lax.shift_left(
                _f16_bits(hb), 16)
            return carry

        lax.fori_loop(0, RN // 32, body, 0, unroll=4)
        pltpu.sync_copy(h_buf, h_out.at[pl.ds(s * (RN // 2), RN // 2)])

    @pl.when(c == 1)
    def _():
        pltpu.sync_copy(deg_sh.at[sl], indeg_out.at[sl])


def _k2_body(ei_hbm, h_hbm, zeros_hbm,
             agg_out,
             agg_sh, h_buf, sidx_a, sidx_b, didx_a, didx_b, msg_a, msg_b,
             ssem_a, ssem_b, dsem_a, dsem_b, scsem_a, scsem_b):
    c = lax.axis_index("c")
    s = lax.axis_index("s")
    sl = pl.ds(s * RN, RN)
    pltpu.sync_copy(zeros_hbm.at[sl], agg_sh.at[sl])
    pltpu.sync_copy(h_hbm, h_buf)
    plsc.subcore_barrier()

    base = (c * NS + s) * EPT2
    sidx = [sidx_a, sidx_b]
    didx = [didx_a, didx_b]
    msg = [msg_a, msg_b]
    ssems = [ssem_a, ssem_b]
    dsems = [dsem_a, dsem_b]
    scsems = [scsem_a, scsem_b]
    sdesc = [None, None]
    ddesc = [None, None]
    scdesc = [None, None]

    sdesc[0] = pltpu.async_copy(
        ei_hbm.at[pl.ds(base, C2)], sidx[0], ssems[0])
    ddesc[0] = pltpu.async_copy(
        ei_hbm.at[pl.ds(E + base, C2)], didx[0], dsems[0])
    for k in range(NCH2):
        b = k % 2
        if k + 1 < NCH2:
            nb = (k + 1) % 2
            off = base + (k + 1) * C2
            sdesc[nb] = pltpu.async_copy(
                ei_hbm.at[pl.ds(off, C2)], sidx[nb], ssems[nb])
            ddesc[nb] = pltpu.async_copy(
                ei_hbm.at[pl.ds(E + off, C2)], didx[nb], dsems[nb])
        sdesc[b].wait()
        if scdesc[b] is not None:
            scdesc[b].wait()

        def gather(i, carry):
            v = pl.ds(i * 16, 16)
            n = sidx[b][v]
            iw = lax.shift_left(lax.shift_right_logical(n, 5), 4) | (n & 15)
            g = plsc.load_gather(h_buf, [iw])
            sh = lax.shift_left(lax.shift_right_logical(n, 4) & 1, 4)
            b16 = lax.shift_right_logical(g, sh) & jnp.int32(0xFFFF)
            msg[b][v] = _f16_to_f32(b16)
            return carry

        lax.fori_loop(0, C2 // 16, gather, 0, unroll=4)
        ddesc[b].wait()
        scdesc[b] = pltpu.async_copy(
            msg[b], agg_sh.at[didx[b]], scsems[b], add=True)
    for d in scdesc:
        if d is not None:
            d.wait()
    plsc.subcore_barrier()
    pltpu.sync_copy(agg_sh.at[sl], agg_out.at[c, sl])


def _k3_body(aggp_ref, indeg_ref, w1_ref, b1_ref, wl1_ref, bl1_ref,
             wl2_ref, bl2_ref, out_ref, a_ref, hg_ref, h1_ref):
    a = (aggp_ref[0] + aggp_ref[1]) * lax.rsqrt(
        jnp.maximum(indeg_ref[...], jnp.float32(1.0)))
    a_ref[...] = a
    inv_n = jnp.float32(1.0 / N)
    for j in range(F1):
        w = w1_ref[j]
        b = b1_ref[j]
        colsum = jnp.sum(jnp.maximum(a_ref[...] * w + b, 0.0))
        colsum = colsum - PADC * jnp.maximum(b, 0.0)
        hg_ref[j] = jnp.maximum(colsum * inv_n, 0.0)

    def l1_body(k, carry):
        def inner(j, acc):
            return acc + hg_ref[j] * wl1_ref[k * F1 + j]

        acc = lax.fori_loop(0, F1, inner, bl1_ref[k])
        h1_ref[k] = jnp.maximum(acc, 0.0)
        return carry

    lax.fori_loop(0, F2, l1_body, 0)

    def l2_body(m, carry):
        def inner(k, acc):
            return acc + h1_ref[k] * wl2_ref[m * F2 + k]

        acc = lax.fori_loop(0, F2, inner, bl2_ref[m])
        out_ref[m] = jnp.maximum(acc, 0.0)
        return carry

    lax.fori_loop(0, F3, l2_body, 0)


def kernel(x, edge_index, W1, b1, W_lin1, b_lin1, W_lin2, b_lin2):
    ei = edge_index.astype(jnp.int32).reshape(2 * E)
    xp = jnp.pad(x[:, 0].astype(jnp.float32), (0, PADC))
    zeros = jnp.zeros((NPAD,), jnp.float32)
    ones = jnp.ones((C1,), jnp.float32)

    mesh = plsc.VectorSubcoreMesh(
        core_axis_name="c", subcore_axis_name="s",
        num_cores=NC, num_subcores=NS)

    h, indeg = pl.kernel(
        _k1_body,
        out_type=(
            jax.ShapeDtypeStruct((NPAD // 2,), jnp.int32),
            jax.ShapeDtypeStruct((NPAD,), jnp.float32),
        ),
        mesh=mesh,
        scratch_types=[
            pltpu.VMEM_SHARED((NPAD,), jnp.float32),
            pltpu.VMEM((C1,), jnp.int32),
            pltpu.VMEM((C1,), jnp.int32),
            pltpu.VMEM((C1,), jnp.float32),
            pltpu.VMEM((RN,), jnp.float32),
            pltpu.VMEM((RN,), jnp.float32),
            pltpu.VMEM((RN // 2,), jnp.int32),
            pltpu.SemaphoreType.DMA,
            pltpu.SemaphoreType.DMA,
            pltpu.SemaphoreType.DMA,
            pltpu.SemaphoreType.DMA,
        ],
        compiler_params=pltpu.CompilerParams(needs_layout_passes=False),
        name="gcn_degrees",
    )(ei, xp, zeros, ones)

    aggp = pl.kernel(
        _k2_body,
        out_type=jax.ShapeDtypeStruct((NC, NPAD), jnp.float32),
        mesh=mesh,
        scratch_types=[
            pltpu.VMEM_SHARED((NPAD,), jnp.float32),
            pltpu.VMEM((NPAD // 2,), jnp.int32),
            pltpu.VMEM((C2,), jnp.int32),
            pltpu.VMEM((C2,), jnp.int32),
            pltpu.VMEM((C2,), jnp.int32),
            pltpu.VMEM((C2,), jnp.int32),
            pltpu.VMEM((C2,), jnp.float32),
            pltpu.VMEM((C2,), jnp.float32),
            pltpu.SemaphoreType.DMA,
            pltpu.SemaphoreType.DMA,
            pltpu.SemaphoreType.DMA,
            pltpu.SemaphoreType.DMA,
            pltpu.SemaphoreType.DMA,
            pltpu.SemaphoreType.DMA,
        ],
        compiler_params=pltpu.CompilerParams(needs_layout_passes=False),
        name="gcn_messages",
    )(ei, h, zeros)

    out = pl.pallas_call(
        _k3_body,
        out_shape=jax.ShapeDtypeStruct((F3,), jnp.float32),
        in_specs=[
            pl.BlockSpec(memory_space=pltpu.VMEM),
            pl.BlockSpec(memory_space=pltpu.VMEM),
            pl.BlockSpec(memory_space=pltpu.SMEM),
            pl.BlockSpec(memory_space=pltpu.SMEM),
            pl.BlockSpec(memory_space=pltpu.SMEM),
            pl.BlockSpec(memory_space=pltpu.SMEM),
            pl.BlockSpec(memory_space=pltpu.SMEM),
            pl.BlockSpec(memory_space=pltpu.SMEM),
        ],
        out_specs=pl.BlockSpec(memory_space=pltpu.SMEM),
        scratch_shapes=[
            pltpu.VMEM((NPAD // 128, 128), jnp.float32),
            pltpu.SMEM((F1,), jnp.float32),
            pltpu.SMEM((F2,), jnp.float32),
        ],
        name="gcn_head",
    )(
        aggp.reshape(NC, NPAD // 128, 128),
        indeg.reshape(NPAD // 128, 128),
        W1.reshape(F1).astype(jnp.float32),
        b1.astype(jnp.float32),
        W_lin1.reshape(F2 * F1).astype(jnp.float32),
        b_lin1.astype(jnp.float32),
        W_lin2.reshape(F3 * F2).astype(jnp.float32),
        b_lin2.astype(jnp.float32),
    )
    return out.reshape(1, F3)


# speedup vs baseline: 1.5842x; 1.5842x over previous
import jax
import jax.numpy as jnp
from jax import lax
from jax.experimental import pallas as pl
from jax.experimental.pallas import tpu as pltpu
from jax.experimental.pallas import tpu_sc as plsc

N = 100000
E = 6400000
NC = 2
NS = 16
NPAD = 100352
RN = NPAD // NS
PADC = NPAD - N

EPT1 = E // NS
C1 = 3200
NCH1 = EPT1 // C1

C2 = 6400
NCH2 = E // (NC * NS * C2)
NXTRA = E // C2 - NC * NS * NCH2

F1 = 60
F2 = 30
F3 = 10


def _f16_bits(h):
    b = lax.bitcast_convert_type(h, jnp.int32)
    sgn = lax.shift_right_logical(b, 16) & jnp.int32(0x8000)
    mag = b & jnp.int32(0x7FFFFFFF)
    lsb = lax.shift_right_logical(mag, 13) & 1
    rounded = mag + jnp.int32(0xFFF) + lsb
    r16 = lax.shift_right_logical(rounded, 13) - jnp.int32((127 - 15) << 10)
    r16 = jnp.where(r16 <= 0, jnp.int32(0), r16)
    return sgn | r16


def _f16_to_f32(b16):
    sgn = lax.shift_left(b16 & jnp.int32(0x8000), 16)
    rest = b16 & jnp.int32(0x7FFF)
    f32b = jnp.where(rest == 0, jnp.int32(0),
                     lax.shift_left(rest + jnp.int32(112 << 10), 13))
    return lax.bitcast_convert_type(sgn | f32b, jnp.float32)


def _rsqrt_newton(d):
    i = lax.bitcast_convert_type(d, jnp.int32)
    i = jnp.int32(0x5F3759DF) - lax.shift_right_logical(i, 1)
    y = lax.bitcast_convert_type(i, jnp.float32)
    for _ in range(3):
        y = y * (jnp.float32(1.5) - jnp.float32(0.5) * d * y * y)
    return y


def _k1_body(ei_hbm, x_hbm, zeros_hbm, ones_hbm,
             h_out, indeg_out,
             deg_sh, idx_a, idx_b, row_a, row_b, ones_buf, deg_buf, x_buf,
             h_buf, dma_sem_a, dma_sem_b, sc_sem_a, sc_sem_b):
    c = lax.axis_index("c")
    s = lax.axis_index("s")
    sl = pl.ds(s * RN, RN)
    pltpu.sync_copy(zeros_hbm.at[sl], deg_sh.at[sl])

    pltpu.sync_copy(ones_hbm, ones_buf)
    plsc.subcore_barrier()

    base = s * EPT1
    idx_bufs = [idx_a, idx_b]
    row_bufs = [row_a, row_b]
    dma_sems = [dma_sem_a, dma_sem_b]
    sc_sems = [sc_sem_a, sc_sem_b]
    dma_descs = [None, None]
    sc_descs = [None, None]
    dma_descs[0] = pltpu.async_copy(
        ei_hbm.at[:, pl.ds(base, C1)], idx_bufs[0], dma_sems[0])
    for k in range(NCH1):
        b = k % 2
        if k + 1 < NCH1:
            nb = (k + 1) % 2
            dma_descs[nb] = pltpu.async_copy(
                ei_hbm.at[:, pl.ds(base + (k + 1) * C1, C1)],
                idx_bufs[nb], dma_sems[nb])
        dma_descs[b].wait()
        if sc_descs[b] is not None:
            sc_descs[b].wait()
        rb = row_bufs[b]
        i2 = idx_bufs[b]

        @plsc.parallel_loop(0, C1 // 16, unroll=8)
        def rowcopy(i):
            v = pl.ds(i * 16, 16)
            rb[v] = i2[c, v]

        sc_descs[b] = pltpu.async_copy(
            ones_buf, deg_sh.at[row_bufs[b]], sc_sems[b], add=True)
    for d in sc_descs:
        if d is not None:
            d.wait()
    plsc.subcore_barrier()

    @pl.when(c == 0)
    def _():
        pltpu.sync_copy(deg_sh.at[sl], deg_buf)
        pltpu.sync_copy(x_hbm.at[sl], x_buf)

        def body(i, carry):
            va = pl.ds(i * 32, 16)
            vb = pl.ds(i * 32 + 16, 16)
            da = jnp.maximum(deg_buf[va], jnp.float32(1.0))
            db = jnp.maximum(deg_buf[vb], jnp.float32(1.0))
            ha = x_buf[va] * _rsqrt_newton(da)
            hb = x_buf[vb] * _rsqrt_newton(db)
            h_buf[pl.ds(i * 16, 16)] = _f16_bits(ha) | lax.shift_left(
                _f16_bits(hb), 16)
            return carry

        lax.fori_loop(0, RN // 32, body, 0, unroll=4)
        pltpu.sync_copy(h_buf, h_out.at[pl.ds(s * (RN // 2), RN // 2)])

    @pl.when(c == 1)
    def _():
        pltpu.sync_copy(deg_sh.at[sl], indeg_out.at[sl])


def _k2_body(ei_hbm, h_hbm, zeros_hbm,
             agg_out,
             agg_sh, h_buf, eidx_a, eidx_b, didx_a, didx_b, msg_a, msg_b,
             esem_a, esem_b, scsem_a, scsem_b):
    c = lax.axis_index("c")
    s = lax.axis_index("s")
    sl = pl.ds(s * RN, RN)
    pltpu.sync_copy(zeros_hbm.at[sl], agg_sh.at[sl])
    pltpu.sync_copy(h_hbm, h_buf)
    plsc.subcore_barrier()

    w = c * NS + s
    eidx = [eidx_a, eidx_b]
    didx = [didx_a, didx_b]
    msg = [msg_a, msg_b]
    esems = [esem_a, esem_b]
    scsems = [scsem_a, scsem_b]
    edesc = [None, None]
    scdesc = [None, None]

    def chunk_off(k):
        extra = (NC * NS * NCH2 + w) * C2
        return jnp.where(k == NCH2,
                         jnp.where(w < NXTRA, extra, 0),
                         (w * NCH2 + k) * C2)

    def do_gather(b):
        @plsc.parallel_loop(0, C2 // 16, unroll=4)
        def gather(i):
            v = pl.ds(i * 16, 16)
            n = eidx[b][0, v]
            iw = lax.shift_left(lax.shift_right_logical(n, 5), 4) | (n & 15)
            g = plsc.load_gather(h_buf, [iw])
            sh = lax.shift_left(lax.shift_right_logical(n, 4) & 1, 4)
            b16 = lax.shift_right_logical(g, sh) & jnp.int32(0xFFFF)
            msg[b][v] = _f16_to_f32(b16)
            didx[b][v] = eidx[b][1, v]

    nch = NCH2 + 1
    edesc[0] = pltpu.async_copy(
        ei_hbm.at[:, pl.ds(chunk_off(0), C2)], eidx[0], esems[0])
    for k in range(nch):
        b = k % 2
        if k + 1 < nch:
            nb = (k + 1) % 2
            edesc[nb] = pltpu.async_copy(
                ei_hbm.at[:, pl.ds(chunk_off(k + 1), C2)],
                eidx[nb], esems[nb])
        edesc[b].wait()
        if scdesc[b] is not None:
            scdesc[b].wait()
            scdesc[b] = None
        if k < NCH2:
            do_gather(b)
            scdesc[b] = pltpu.async_copy(
                msg[b], agg_sh.at[didx[b]], scsems[b], add=True)
        else:
            @pl.when(w < NXTRA)
            def _():
                do_gather(b)
                pltpu.async_copy(
                    msg[b], agg_sh.at[didx[b]], scsems[b],
                    add=True).wait()
    for d in scdesc:
        if d is not None:
            d.wait()
    plsc.subcore_barrier()
    pltpu.sync_copy(agg_sh.at[sl], agg_out.at[c, sl])


def _k3_body(aggp_ref, indeg_ref, w1_ref, b1_ref, wl1_ref, bl1_ref,
             wl2_ref, bl2_ref, out_ref, a_ref, hg_ref, h1_ref):
    a = (aggp_ref[0] + aggp_ref[1]) * lax.rsqrt(
        jnp.maximum(indeg_ref[...], jnp.float32(1.0)))
    a_ref[...] = a
    inv_n = jnp.float32(1.0 / N)
    for j in range(F1):
        w = w1_ref[j]
        b = b1_ref[j]
        colsum = jnp.sum(jnp.maximum(a_ref[...] * w + b, 0.0))
        colsum = colsum - PADC * jnp.maximum(b, 0.0)
        hg_ref[j] = jnp.maximum(colsum * inv_n, 0.0)

    def l1_body(k, carry):
        def inner(j, acc):
            return acc + hg_ref[j] * wl1_ref[k * F1 + j]

        acc = lax.fori_loop(0, F1, inner, bl1_ref[k])
        h1_ref[k] = jnp.maximum(acc, 0.0)
        return carry

    lax.fori_loop(0, F2, l1_body, 0)

    def l2_body(m, carry):
        def inner(k, acc):
            return acc + h1_ref[k] * wl2_ref[m * F2 + k]

        acc = lax.fori_loop(0, F2, inner, bl2_ref[m])
        out_ref[m] = jnp.maximum(acc, 0.0)
        return carry

    lax.fori_loop(0, F3, l2_body, 0)


def kernel(x, edge_index, W1, b1, W_lin1, b_lin1, W_lin2, b_lin2):
    ei = edge_index.astype(jnp.int32)
    xp = jnp.pad(x[:, 0].astype(jnp.float32), (0, PADC))
    zeros = jnp.zeros((NPAD,), jnp.float32)
    ones = jnp.ones((C1,), jnp.float32)

    mesh = plsc.VectorSubcoreMesh(
        core_axis_name="c", subcore_axis_name="s",
        num_cores=NC, num_subcores=NS)

    h, indeg = pl.kernel(
        _k1_body,
        out_type=(
            jax.ShapeDtypeStruct((NPAD // 2,), jnp.int32),
            jax.ShapeDtypeStruct((NPAD,), jnp.float32),
        ),
        mesh=mesh,
        scratch_types=[
            pltpu.VMEM_SHARED((NPAD,), jnp.float32),
            pltpu.VMEM((2, C1), jnp.int32),
            pltpu.VMEM((2, C1), jnp.int32),
            pltpu.VMEM((C1,), jnp.int32),
            pltpu.VMEM((C1,), jnp.int32),
            pltpu.VMEM((C1,), jnp.float32),
            pltpu.VMEM((RN,), jnp.float32),
            pltpu.VMEM((RN,), jnp.float32),
            pltpu.VMEM((RN // 2,), jnp.int32),
            pltpu.SemaphoreType.DMA,
            pltpu.SemaphoreType.DMA,
            pltpu.SemaphoreType.DMA,
            pltpu.SemaphoreType.DMA,
        ],
        compiler_params=pltpu.CompilerParams(needs_layout_passes=False),
        name="gcn_degrees",
    )(ei, xp, zeros, ones)

    aggp = pl.kernel(
        _k2_body,
        out_type=jax.ShapeDtypeStruct((NC, NPAD), jnp.float32),
        mesh=mesh,
        scratch_types=[
            pltpu.VMEM_SHARED((NPAD,), jnp.float32),
            pltpu.VMEM((NPAD // 2,), jnp.int32),
            pltpu.VMEM((2, C2), jnp.int32),
            pltpu.VMEM((2, C2), jnp.int32),
            pltpu.VMEM((C2,), jnp.int32),
            pltpu.VMEM((C2,), jnp.int32),
            pltpu.VMEM((C2,), jnp.float32),
            pltpu.VMEM((C2,), jnp.float32),
            pltpu.SemaphoreType.DMA,
            pltpu.SemaphoreType.DMA,
            pltpu.SemaphoreType.DMA,
            pltpu.SemaphoreType.DMA,
        ],
        compiler_params=pltpu.CompilerParams(needs_layout_passes=False),
        name="gcn_messages",
    )(ei, h, zeros)

    out = pl.pallas_call(
        _k3_body,
        out_shape=jax.ShapeDtypeStruct((F3,), jnp.float32),
        in_specs=[
            pl.BlockSpec(memory_space=pltpu.VMEM),
            pl.BlockSpec(memory_space=pltpu.VMEM),
            pl.BlockSpec(memory_space=pltpu.SMEM),
            pl.BlockSpec(memory_space=pltpu.SMEM),
            pl.BlockSpec(memory_space=pltpu.SMEM),
            pl.BlockSpec(memory_space=pltpu.SMEM),
            pl.BlockSpec(memory_space=pltpu.SMEM),
            pl.BlockSpec(memory_space=pltpu.SMEM),
        ],
        out_specs=pl.BlockSpec(memory_space=pltpu.SMEM),
        scratch_shapes=[
            pltpu.VMEM((NPAD // 128, 128), jnp.float32),
            pltpu.SMEM((F1,), jnp.float32),
            pltpu.SMEM((F2,), jnp.float32),
        ],
        name="gcn_head",
    )(
        aggp.reshape(NC, NPAD // 128, 128),
        indeg.reshape(NPAD // 128, 128),
        W1.reshape(F1).astype(jnp.float32),
        b1.astype(jnp.float32),
        W_lin1.reshape(F2 * F1).astype(jnp.float32),
        b_lin1.astype(jnp.float32),
        W_lin2.reshape(F3 * F2).astype(jnp.float32),
        b_lin2.astype(jnp.float32),
    )
    return out.reshape(1, F3)


# speedup vs baseline: 1.6076x; 1.0147x over previous
import jax
import jax.numpy as jnp
from jax import lax
from jax.experimental import pallas as pl
from jax.experimental.pallas import tpu as pltpu
from jax.experimental.pallas import tpu_sc as plsc

N = 100000
E = 6400000
NC = 2
NS = 16
NPAD = 100352
RN = NPAD // NS
PADC = NPAD - N

EPT1 = E // NS
C1 = 16000
NCH1 = EPT1 // C1

C2 = 6400
NCH2 = E // (NC * NS * C2)
NXTRA = E // C2 - NC * NS * NCH2

F1 = 60
F2 = 30
F3 = 10


def _f16_bits(h):
    b = lax.bitcast_convert_type(h, jnp.int32)
    sgn = lax.shift_right_logical(b, 16) & jnp.int32(0x8000)
    mag = b & jnp.int32(0x7FFFFFFF)
    lsb = lax.shift_right_logical(mag, 13) & 1
    rounded = mag + jnp.int32(0xFFF) + lsb
    r16 = lax.shift_right_logical(rounded, 13) - jnp.int32((127 - 15) << 10)
    r16 = jnp.where(r16 <= 0, jnp.int32(0), r16)
    return sgn | r16


def _f16_to_f32(b16):
    sgn = lax.shift_left(b16 & jnp.int32(0x8000), 16)
    rest = b16 & jnp.int32(0x7FFF)
    f32b = jnp.where(rest == 0, jnp.int32(0),
                     lax.shift_left(rest + jnp.int32(112 << 10), 13))
    return lax.bitcast_convert_type(sgn | f32b, jnp.float32)


def _rsqrt_newton(d):
    i = lax.bitcast_convert_type(d, jnp.int32)
    i = jnp.int32(0x5F3759DF) - lax.shift_right_logical(i, 1)
    y = lax.bitcast_convert_type(i, jnp.float32)
    for _ in range(3):
        y = y * (jnp.float32(1.5) - jnp.float32(0.5) * d * y * y)
    return y


def _k1_body(ei_hbm, x_hbm, zeros_hbm, ones_hbm,
             h_out, indeg_out,
             deg_sh, idx_a, idx_b, row_a, row_b, ones_buf, deg_buf,
             dma_sem_a, dma_sem_b, sc_sem_a, sc_sem_b):
    c = lax.axis_index("c")
    s = lax.axis_index("s")
    sl = pl.ds(s * RN, RN)
    pltpu.sync_copy(zeros_hbm.at[sl], deg_sh.at[sl])

    pltpu.sync_copy(ones_hbm, ones_buf)
    plsc.subcore_barrier()

    base = s * EPT1
    idx_bufs = [idx_a, idx_b]
    row_bufs = [row_a, row_b]
    dma_sems = [dma_sem_a, dma_sem_b]
    sc_sems = [sc_sem_a, sc_sem_b]
    dma_descs = [None, None]
    sc_descs = [None, None]
    dma_descs[0] = pltpu.async_copy(
        ei_hbm.at[:, pl.ds(base, C1)], idx_bufs[0], dma_sems[0])
    for k in range(NCH1):
        b = k % 2
        if k + 1 < NCH1:
            nb = (k + 1) % 2
            dma_descs[nb] = pltpu.async_copy(
                ei_hbm.at[:, pl.ds(base + (k + 1) * C1, C1)],
                idx_bufs[nb], dma_sems[nb])
        dma_descs[b].wait()
        if sc_descs[b] is not None:
            sc_descs[b].wait()
        rb = row_bufs[b]
        i2 = idx_bufs[b]

        @plsc.parallel_loop(0, C1 // 16, unroll=8)
        def rowcopy(i):
            v = pl.ds(i * 16, 16)
            rb[v] = i2[c, v]

        sc_descs[b] = pltpu.async_copy(
            ones_buf, deg_sh.at[row_bufs[b]], sc_sems[b], add=True)
    for d in sc_descs:
        if d is not None:
            d.wait()
    plsc.subcore_barrier()

    @pl.when(c == 0)
    def _():
        pltpu.sync_copy(deg_sh.at[sl], deg_buf)
        pltpu.sync_copy(x_hbm.at[sl], row_b.at[pl.ds(0, RN)])

        def fbuf(buf, v):
            return lax.bitcast_convert_type(buf[v], jnp.float32)

        def body(i, carry):
            va = pl.ds(i * 32, 16)
            vb = pl.ds(i * 32 + 16, 16)
            da = jnp.maximum(deg_buf[va], jnp.float32(1.0))
            db = jnp.maximum(deg_buf[vb], jnp.float32(1.0))
            ha = fbuf(row_b, va) * _rsqrt_newton(da)
            hb = fbuf(row_b, vb) * _rsqrt_newton(db)
            row_a[pl.ds(8000 + i * 16, 16)] = _f16_bits(ha) | lax.shift_left(
                _f16_bits(hb), 16)
            return carry

        lax.fori_loop(0, RN // 32, body, 0, unroll=4)
        pltpu.sync_copy(row_a.at[pl.ds(8000, RN // 2)],
                        h_out.at[pl.ds(s * (RN // 2), RN // 2)])

    @pl.when(c == 1)
    def _():
        pltpu.sync_copy(deg_sh.at[sl], indeg_out.at[sl])


def _k2_body(ei_hbm, h_hbm, zeros_hbm,
             agg_out,
             agg_sh, h_buf, eidx_a, eidx_b, didx_a, didx_b, msg_a, msg_b,
             esem_a, esem_b, scsem_a, scsem_b):
    c = lax.axis_index("c")
    s = lax.axis_index("s")
    sl = pl.ds(s * RN, RN)
    pltpu.sync_copy(zeros_hbm.at[sl], agg_sh.at[sl])
    pltpu.sync_copy(h_hbm, h_buf)
    plsc.subcore_barrier()

    w = c * NS + s
    eidx = [eidx_a, eidx_b]
    didx = [didx_a, didx_b]
    msg = [msg_a, msg_b]
    esems = [esem_a, esem_b]
    scsems = [scsem_a, scsem_b]
    edesc = [None, None]
    scdesc = [None, None]

    def chunk_off(k):
        extra = (NC * NS * NCH2 + w) * C2
        return jnp.where(k == NCH2,
                         jnp.where(w < NXTRA, extra, 0),
                         (w * NCH2 + k) * C2)

    def do_gather(b):
        @plsc.parallel_loop(0, C2 // 16, unroll=4)
        def gather(i):
            v = pl.ds(i * 16, 16)
            n = eidx[b][0, v]
            iw = lax.shift_left(lax.shift_right_logical(n, 5), 4) | (n & 15)
            g = plsc.load_gather(h_buf, [iw])
            sh = lax.shift_left(lax.shift_right_logical(n, 4) & 1, 4)
            b16 = lax.shift_right_logical(g, sh) & jnp.int32(0xFFFF)
            msg[b][v] = _f16_to_f32(b16)
            didx[b][v] = eidx[b][1, v]

    nch = NCH2 + 1
    edesc[0] = pltpu.async_copy(
        ei_hbm.at[:, pl.ds(chunk_off(0), C2)], eidx[0], esems[0])
    for k in range(nch):
        b = k % 2
        if k + 1 < nch:
            nb = (k + 1) % 2
            edesc[nb] = pltpu.async_copy(
                ei_hbm.at[:, pl.ds(chunk_off(k + 1), C2)],
                eidx[nb], esems[nb])
        edesc[b].wait()
        if scdesc[b] is not None:
            scdesc[b].wait()
            scdesc[b] = None
        if k < NCH2:
            do_gather(b)
            scdesc[b] = pltpu.async_copy(
                msg[b], agg_sh.at[didx[b]], scsems[b], add=True)
        else:
            @pl.when(w < NXTRA)
            def _():
                do_gather(b)
                pltpu.async_copy(
                    msg[b], agg_sh.at[didx[b]], scsems[b],
                    add=True).wait()
    for d in scdesc:
        if d is not None:
            d.wait()
    plsc.subcore_barrier()
    pltpu.sync_copy(agg_sh.at[sl], agg_out.at[c, sl])


def _k3_body(aggp_ref, indeg_ref, w1_ref, b1_ref, wl1_ref, bl1_ref,
             wl2_ref, bl2_ref, out_ref, a_ref, hg_ref, h1_ref):
    a = (aggp_ref[0] + aggp_ref[1]) * lax.rsqrt(
        jnp.maximum(indeg_ref[...], jnp.float32(1.0)))
    a_ref[...] = a
    inv_n = jnp.float32(1.0 / N)
    for j in range(F1):
        w = w1_ref[j]
        b = b1_ref[j]
        colsum = jnp.sum(jnp.maximum(a_ref[...] * w + b, 0.0))
        colsum = colsum - PADC * jnp.maximum(b, 0.0)
        hg_ref[j] = jnp.maximum(colsum * inv_n, 0.0)

    def l1_body(k, carry):
        def inner(j, acc):
            return acc + hg_ref[j] * wl1_ref[k * F1 + j]

        acc = lax.fori_loop(0, F1, inner, bl1_ref[k])
        h1_ref[k] = jnp.maximum(acc, 0.0)
        return carry

    lax.fori_loop(0, F2, l1_body, 0)

    def l2_body(m, carry):
        def inner(k, acc):
            return acc + h1_ref[k] * wl2_ref[m * F2 + k]

        acc = lax.fori_loop(0, F2, inner, bl2_ref[m])
        out_ref[m] = jnp.maximum(acc, 0.0)
        return carry

    lax.fori_loop(0, F3, l2_body, 0)


def kernel(x, edge_index, W1, b1, W_lin1, b_lin1, W_lin2, b_lin2):
    ei = edge_index.astype(jnp.int32)
    xp = jnp.pad(x[:, 0].astype(jnp.float32), (0, PADC))
    xp_i = lax.bitcast_convert_type(xp, jnp.int32)
    zeros = jnp.zeros((NPAD,), jnp.float32)
    ones = jnp.ones((C1,), jnp.float32)

    mesh = plsc.VectorSubcoreMesh(
        core_axis_name="c", subcore_axis_name="s",
        num_cores=NC, num_subcores=NS)

    h, indeg = pl.kernel(
        _k1_body,
        out_type=(
            jax.ShapeDtypeStruct((NPAD // 2,), jnp.int32),
            jax.ShapeDtypeStruct((NPAD,), jnp.float32),
        ),
        mesh=mesh,
        scratch_types=[
            pltpu.VMEM_SHARED((NPAD,), jnp.float32),
            pltpu.VMEM((2, C1), jnp.int32),
            pltpu.VMEM((2, C1), jnp.int32),
            pltpu.VMEM((C1,), jnp.int32),
            pltpu.VMEM((C1,), jnp.int32),
            pltpu.VMEM((C1,), jnp.float32),
            pltpu.VMEM((RN,), jnp.float32),
            pltpu.SemaphoreType.DMA,
            pltpu.SemaphoreType.DMA,
            pltpu.SemaphoreType.DMA,
            pltpu.SemaphoreType.DMA,
        ],
        compiler_params=pltpu.CompilerParams(needs_layout_passes=False),
        name="gcn_degrees",
    )(ei, xp_i, zeros, ones)

    aggp = pl.kernel(
        _k2_body,
        out_type=jax.ShapeDtypeStruct((NC, NPAD), jnp.float32),
        mesh=mesh,
        scratch_types=[
            pltpu.VMEM_SHARED((NPAD,), jnp.float32),
            pltpu.VMEM((NPAD // 2,), jnp.int32),
            pltpu.VMEM((2, C2), jnp.int32),
            pltpu.VMEM((2, C2), jnp.int32),
            pltpu.VMEM((C2,), jnp.int32),
            pltpu.VMEM((C2,), jnp.int32),
            pltpu.VMEM((C2,), jnp.float32),
            pltpu.VMEM((C2,), jnp.float32),
            pltpu.SemaphoreType.DMA,
            pltpu.SemaphoreType.DMA,
            pltpu.SemaphoreType.DMA,
            pltpu.SemaphoreType.DMA,
        ],
        compiler_params=pltpu.CompilerParams(needs_layout_passes=False),
        name="gcn_messages",
    )(ei, h, zeros)

    out = pl.pallas_call(
        _k3_body,
        out_shape=jax.ShapeDtypeStruct((F3,), jnp.float32),
        in_specs=[
            pl.BlockSpec(memory_space=pltpu.VMEM),
            pl.BlockSpec(memory_space=pltpu.VMEM),
            pl.BlockSpec(memory_space=pltpu.SMEM),
            pl.BlockSpec(memory_space=pltpu.SMEM),
            pl.BlockSpec(memory_space=pltpu.SMEM),
            pl.BlockSpec(memory_space=pltpu.SMEM),
            pl.BlockSpec(memory_space=pltpu.SMEM),
            pl.BlockSpec(memory_space=pltpu.SMEM),
        ],
        out_specs=pl.BlockSpec(memory_space=pltpu.SMEM),
        scratch_shapes=[
            pltpu.VMEM((NPAD // 128, 128), jnp.float32),
            pltpu.SMEM((F1,), jnp.float32),
            pltpu.SMEM((F2,), jnp.float32),
        ],
        name="gcn_head",
    )(
        aggp.reshape(NC, NPAD // 128, 128),
        indeg.reshape(NPAD // 128, 128),
        W1.reshape(F1).astype(jnp.float32),
        b1.astype(jnp.float32),
        W_lin1.reshape(F2 * F1).astype(jnp.float32),
        b_lin1.astype(jnp.float32),
        W_lin2.reshape(F3 * F2).astype(jnp.float32),
        b_lin2.astype(jnp.float32),
    )
    return out.reshape(1, F3)


# speedup vs baseline: 1.6369x; 1.0182x over previous
import jax
import jax.numpy as jnp
from jax import lax
from jax.experimental import pallas as pl
from jax.experimental.pallas import tpu as pltpu
from jax.experimental.pallas import tpu_sc as plsc

N = 100000
E = 6400000
NC = 2
NS = 16
NPAD = 100352
RN = NPAD // NS
PADC = NPAD - N

EPT1 = E // NS
C1 = 16000
NCH1 = EPT1 // C1

C2 = 6400
NCH2 = E // (NC * NS * C2)
NXTRA = E // C2 - NC * NS * NCH2

F1 = 60
F2 = 30
F3 = 10


def _f16_bits(h):
    b = lax.bitcast_convert_type(h, jnp.int32)
    sgn = lax.shift_right_logical(b, 16) & jnp.int32(0x8000)
    mag = b & jnp.int32(0x7FFFFFFF)
    lsb = lax.shift_right_logical(mag, 13) & 1
    rounded = mag + jnp.int32(0xFFF) + lsb
    r16 = lax.shift_right_logical(rounded, 13) - jnp.int32((127 - 15) << 10)
    r16 = jnp.where(r16 <= 0, jnp.int32(0), r16)
    return sgn | r16


def _f16_to_f32(b16):
    sgn = lax.shift_left(b16 & jnp.int32(0x8000), 16)
    rest = b16 & jnp.int32(0x7FFF)
    f32b = jnp.where(rest == 0, jnp.int32(0),
                     lax.shift_left(rest + jnp.int32(112 << 10), 13))
    return lax.bitcast_convert_type(sgn | f32b, jnp.float32)


def _rsqrt_newton(d):
    i = lax.bitcast_convert_type(d, jnp.int32)
    i = jnp.int32(0x5F3759DF) - lax.shift_right_logical(i, 1)
    y = lax.bitcast_convert_type(i, jnp.float32)
    for _ in range(3):
        y = y * (jnp.float32(1.5) - jnp.float32(0.5) * d * y * y)
    return y


def _k1_body(ei_hbm, x_hbm, zeros_hbm, ones_hbm,
             h_out, indeg_out,
             deg_sh, idx_a, idx_b, row_a, row_b, ones_buf, deg_buf,
             dma_sem_a, dma_sem_b, sc_sem_a, sc_sem_b):
    c = lax.axis_index("c")
    s = lax.axis_index("s")
    sl = pl.ds(s * RN, RN)
    base = s * EPT1
    idx_bufs = [idx_a, idx_b]
    row_bufs = [row_a, row_b]
    dma_sems = [dma_sem_a, dma_sem_b]
    sc_sems = [sc_sem_a, sc_sem_b]
    dma_descs = [None, None]
    sc_descs = [None, None]
    dma_descs[0] = pltpu.async_copy(
        ei_hbm.at[:, pl.ds(base, C1)], idx_bufs[0], dma_sems[0])
    dma_descs[1] = pltpu.async_copy(
        ei_hbm.at[:, pl.ds(base + C1, C1)], idx_bufs[1], dma_sems[1])
    pltpu.sync_copy(zeros_hbm.at[sl], deg_sh.at[sl])
    pltpu.sync_copy(ones_hbm, ones_buf)
    plsc.subcore_barrier()
    for k in range(NCH1):
        b = k % 2
        dma_descs[b].wait()
        if sc_descs[b] is not None:
            sc_descs[b].wait()
        rb = row_bufs[b]
        i2 = idx_bufs[b]

        @plsc.parallel_loop(0, C1 // 16, unroll=8)
        def rowcopy(i):
            v = pl.ds(i * 16, 16)
            rb[v] = i2[c, v]

        if k + 2 < NCH1:
            dma_descs[b] = pltpu.async_copy(
                ei_hbm.at[:, pl.ds(base + (k + 2) * C1, C1)],
                idx_bufs[b], dma_sems[b])
        sc_descs[b] = pltpu.async_copy(
            ones_buf, deg_sh.at[row_bufs[b]], sc_sems[b], add=True)
    for d in sc_descs:
        if d is not None:
            d.wait()
    plsc.subcore_barrier()

    @pl.when(c == 0)
    def _():
        pltpu.sync_copy(deg_sh.at[sl], deg_buf)
        pltpu.sync_copy(x_hbm.at[sl], row_b.at[pl.ds(0, RN)])

        def fbuf(buf, v):
            return lax.bitcast_convert_type(buf[v], jnp.float32)

        def body(i, carry):
            va = pl.ds(i * 32, 16)
            vb = pl.ds(i * 32 + 16, 16)
            da = jnp.maximum(deg_buf[va], jnp.float32(1.0))
            db = jnp.maximum(deg_buf[vb], jnp.float32(1.0))
            ha = fbuf(row_b, va) * _rsqrt_newton(da)
            hb = fbuf(row_b, vb) * _rsqrt_newton(db)
            row_a[pl.ds(8000 + i * 16, 16)] = _f16_bits(ha) | lax.shift_left(
                _f16_bits(hb), 16)
            return carry

        lax.fori_loop(0, RN // 32, body, 0, unroll=4)
        pltpu.sync_copy(row_a.at[pl.ds(8000, RN // 2)],
                        h_out.at[pl.ds(s * (RN // 2), RN // 2)])

    @pl.when(c == 1)
    def _():
        pltpu.sync_copy(deg_sh.at[sl], indeg_out.at[sl])


def _k2_body(ei_hbm, h_hbm, zeros_hbm,
             agg_out,
             agg_sh, h_buf, eidx_a, eidx_b, didx_a, didx_b, msg_a, msg_b,
             esem_a, esem_b, scsem_a, scsem_b):
    c = lax.axis_index("c")
    s = lax.axis_index("s")
    sl = pl.ds(s * RN, RN)
    w = c * NS + s
    eidx = [eidx_a, eidx_b]
    didx = [didx_a, didx_b]
    msg = [msg_a, msg_b]
    esems = [esem_a, esem_b]
    scsems = [scsem_a, scsem_b]
    edesc = [None, None]
    scdesc = [None, None]

    def chunk_off(k):
        extra = (NC * NS * NCH2 + w) * C2
        return jnp.where(k == NCH2,
                         jnp.where(w < NXTRA, extra, 0),
                         (w * NCH2 + k) * C2)

    def do_gather(b):
        @plsc.parallel_loop(0, C2 // 16, unroll=4)
        def gather(i):
            v = pl.ds(i * 16, 16)
            n = eidx[b][0, v]
            iw = lax.shift_left(lax.shift_right_logical(n, 5), 4) | (n & 15)
            g = plsc.load_gather(h_buf, [iw])
            sh = lax.shift_left(lax.shift_right_logical(n, 4) & 1, 4)
            b16 = lax.shift_right_logical(g, sh) & jnp.int32(0xFFFF)
            msg[b][v] = _f16_to_f32(b16)
            didx[b][v] = eidx[b][1, v]

    nch = NCH2 + 1
    edesc[0] = pltpu.async_copy(
        ei_hbm.at[:, pl.ds(chunk_off(0), C2)], eidx[0], esems[0])
    edesc[1] = pltpu.async_copy(
        ei_hbm.at[:, pl.ds(chunk_off(1), C2)], eidx[1], esems[1])
    pltpu.sync_copy(zeros_hbm.at[sl], agg_sh.at[sl])
    pltpu.sync_copy(h_hbm, h_buf)
    plsc.subcore_barrier()
    for k in range(nch):
        b = k % 2
        edesc[b].wait()
        if scdesc[b] is not None:
            scdesc[b].wait()
            scdesc[b] = None
        if k < NCH2:
            do_gather(b)
            if k + 2 < nch:
                edesc[b] = pltpu.async_copy(
                    ei_hbm.at[:, pl.ds(chunk_off(k + 2), C2)],
                    eidx[b], esems[b])
            scdesc[b] = pltpu.async_copy(
                msg[b], agg_sh.at[didx[b]], scsems[b], add=True)
        else:
            @pl.when(w < NXTRA)
            def _():
                do_gather(b)
                pltpu.async_copy(
                    msg[b], agg_sh.at[didx[b]], scsems[b],
                    add=True).wait()
    for d in scdesc:
        if d is not None:
            d.wait()
    plsc.subcore_barrier()
    pltpu.sync_copy(agg_sh.at[sl], agg_out.at[c, sl])


def _k3_body(aggp_ref, indeg_ref, w1_ref, b1_ref, wl1_ref, bl1_ref,
             wl2_ref, bl2_ref, out_ref, a_ref, hg_ref, h1_ref):
    a = (aggp_ref[0] + aggp_ref[1]) * lax.rsqrt(
        jnp.maximum(indeg_ref[...], jnp.float32(1.0)))
    a_ref[...] = a
    inv_n = jnp.float32(1.0 / N)
    for j in range(F1):
        w = w1_ref[j]
        b = b1_ref[j]
        colsum = jnp.sum(jnp.maximum(a_ref[...] * w + b, 0.0))
        colsum = colsum - PADC * jnp.maximum(b, 0.0)
        hg_ref[j] = jnp.maximum(colsum * inv_n, 0.0)

    def l1_body(k, carry):
        def inner(j, acc):
            return acc + hg_ref[j] * wl1_ref[k * F1 + j]

        acc = lax.fori_loop(0, F1, inner, bl1_ref[k])
        h1_ref[k] = jnp.maximum(acc, 0.0)
        return carry

    lax.fori_loop(0, F2, l1_body, 0)

    def l2_body(m, carry):
        def inner(k, acc):
            return acc + h1_ref[k] * wl2_ref[m * F2 + k]

        acc = lax.fori_loop(0, F2, inner, bl2_ref[m])
        out_ref[m] = jnp.maximum(acc, 0.0)
        return carry

    lax.fori_loop(0, F3, l2_body, 0)


def kernel(x, edge_index, W1, b1, W_lin1, b_lin1, W_lin2, b_lin2):
    ei = edge_index.astype(jnp.int32)
    xp = jnp.pad(x[:, 0].astype(jnp.float32), (0, PADC))
    xp_i = lax.bitcast_convert_type(xp, jnp.int32)
    zeros = jnp.zeros((NPAD,), jnp.float32)
    ones = jnp.ones((C1,), jnp.float32)

    mesh = plsc.VectorSubcoreMesh(
        core_axis_name="c", subcore_axis_name="s",
        num_cores=NC, num_subcores=NS)

    h, indeg = pl.kernel(
        _k1_body,
        out_type=(
            jax.ShapeDtypeStruct((NPAD // 2,), jnp.int32),
            jax.ShapeDtypeStruct((NPAD,), jnp.float32),
        ),
        mesh=mesh,
        scratch_types=[
            pltpu.VMEM_SHARED((NPAD,), jnp.float32),
            pltpu.VMEM((2, C1), jnp.int32),
            pltpu.VMEM((2, C1), jnp.int32),
            pltpu.VMEM((C1,), jnp.int32),
            pltpu.VMEM((C1,), jnp.int32),
            pltpu.VMEM((C1,), jnp.float32),
            pltpu.VMEM((RN,), jnp.float32),
            pltpu.SemaphoreType.DMA,
            pltpu.SemaphoreType.DMA,
            pltpu.SemaphoreType.DMA,
            pltpu.SemaphoreType.DMA,
        ],
        compiler_params=pltpu.CompilerParams(needs_layout_passes=False),
        name="gcn_degrees",
    )(ei, xp_i, zeros, ones)

    aggp = pl.kernel(
        _k2_body,
        out_type=jax.ShapeDtypeStruct((NC, NPAD), jnp.float32),
        mesh=mesh,
        scratch_types=[
            pltpu.VMEM_SHARED((NPAD,), jnp.float32),
            pltpu.VMEM((NPAD // 2,), jnp.int32),
            pltpu.VMEM((2, C2), jnp.int32),
            pltpu.VMEM((2, C2), jnp.int32),
            pltpu.VMEM((C2,), jnp.int32),
            pltpu.VMEM((C2,), jnp.int32),
            pltpu.VMEM((C2,), jnp.float32),
            pltpu.VMEM((C2,), jnp.float32),
            pltpu.SemaphoreType.DMA,
            pltpu.SemaphoreType.DMA,
            pltpu.SemaphoreType.DMA,
            pltpu.SemaphoreType.DMA,
        ],
        compiler_params=pltpu.CompilerParams(needs_layout_passes=False),
        name="gcn_messages",
    )(ei, h, zeros)

    out = pl.pallas_call(
        _k3_body,
        out_shape=jax.ShapeDtypeStruct((F3,), jnp.float32),
        in_specs=[
            pl.BlockSpec(memory_space=pltpu.VMEM),
            pl.BlockSpec(memory_space=pltpu.VMEM),
            pl.BlockSpec(memory_space=pltpu.SMEM),
            pl.BlockSpec(memory_space=pltpu.SMEM),
            pl.BlockSpec(memory_space=pltpu.SMEM),
            pl.BlockSpec(memory_space=pltpu.SMEM),
            pl.BlockSpec(memory_space=pltpu.SMEM),
            pl.BlockSpec(memory_space=pltpu.SMEM),
        ],
        out_specs=pl.BlockSpec(memory_space=pltpu.SMEM),
        scratch_shapes=[
            pltpu.VMEM((NPAD // 128, 128), jnp.float32),
            pltpu.SMEM((F1,), jnp.float32),
            pltpu.SMEM((F2,), jnp.float32),
        ],
        name="gcn_head",
    )(
        aggp.reshape(NC, NPAD // 128, 128),
        indeg.reshape(NPAD // 128, 128),
        W1.reshape(F1).astype(jnp.float32),
        b1.astype(jnp.float32),
        W_lin1.reshape(F2 * F1).astype(jnp.float32),
        b_lin1.astype(jnp.float32),
        W_lin2.reshape(F3 * F2).astype(jnp.float32),
        b_lin2.astype(jnp.float32),
    )
    return out.reshape(1, F3)
